# baseline (speedup 1.0000x reference)
import functools

import jax
import jax.numpy as jnp
from jax.experimental import pallas as pl
from jax.experimental.pallas import tpu as pltpu

E = 8
TOP_K = 2
D_MODEL = 1024
D_FF = 2048
T = 4096
CAP = int(TOP_K * T / E * 1.25)

NF = 4
BF = D_FF // NF


def _ffn_body(buf_ref, wa_ref, wb_ref, wd_ref, out_ref):
    f = pl.program_id(1)
    xb = buf_ref[...].astype(jnp.bfloat16)
    wa = wa_ref[0].astype(jnp.bfloat16)
    wb = wb_ref[0].astype(jnp.bfloat16)
    a = jnp.dot(xb, wa, preferred_element_type=jnp.float32)
    b = jnp.dot(xb, wb, preferred_element_type=jnp.float32)
    h = (a * jax.nn.sigmoid(a) * b).astype(jnp.bfloat16)
    wd = wd_ref[0].astype(jnp.bfloat16)
    contrib = jnp.dot(h, wd, preferred_element_type=jnp.float32)

    @pl.when(f == 0)
    def _init():
        out_ref[...] = contrib

    @pl.when(f > 0)
    def _acc():
        out_ref[...] += contrib


def _ffn(buf, w_up, w_down):
    return pl.pallas_call(
        _ffn_body,
        grid=(E, NF),
        in_specs=[
            pl.BlockSpec((CAP, D_MODEL), lambda e, f: (e, 0)),
            pl.BlockSpec((1, D_MODEL, BF), lambda e, f: (e, 0, f)),
            pl.BlockSpec((1, D_MODEL, BF), lambda e, f: (e, 0, NF + f)),
            pl.BlockSpec((1, BF, D_MODEL), lambda e, f: (e, f, 0)),
        ],
        out_specs=pl.BlockSpec((CAP, D_MODEL), lambda e, f: (e, 0)),
        out_shape=jax.ShapeDtypeStruct((E * CAP, D_MODEL), jnp.float32),
        compiler_params=pltpu.CompilerParams(
            dimension_semantics=("arbitrary", "arbitrary"),
        ),
    )(buf, w_up, w_up, w_down)


def kernel(x, Wr, w_up, w_down):
    logits = x @ Wr
    probs = jax.nn.softmax(logits, axis=-1)
    topv, topi = jax.lax.top_k(probs, TOP_K)
    topv = topv / jnp.sum(topv, axis=-1, keepdims=True)
    flat_e = topi.reshape(-1)
    flat_w = topv.reshape(-1)
    onehot = jax.nn.one_hot(flat_e, E, dtype=jnp.int32)
    pos_in_e = (jnp.cumsum(onehot, axis=0) * onehot).sum(-1) - 1
    valid = pos_in_e < CAP
    token_idx = jnp.repeat(jnp.arange(T), TOP_K)
    dispatch_idx = flat_e * CAP + jnp.clip(pos_in_e, 0, CAP - 1)
    buf = jnp.zeros((E * CAP, D_MODEL), dtype=x.dtype)
    buf = buf.at[dispatch_idx].add(jnp.where(valid[:, None], x[token_idx], 0.0))
    out = _ffn(buf, w_up, w_down)
    gathered = out[dispatch_idx]
    gathered = jnp.where(valid[:, None], gathered, 0.0) * flat_w[:, None]
    y = jnp.zeros((T, D_MODEL), dtype=x.dtype).at[token_idx].add(gathered)
    return y


# speedup vs baseline: 1.5692x; 1.5692x over previous
import functools

import jax
import jax.numpy as jnp
from jax import lax
from jax.experimental import pallas as pl
from jax.experimental.pallas import tpu as pltpu
from jax.experimental.pallas import tpu_sc as plsc

E = 8
TOP_K = 2
D_MODEL = 1024
D_FF = 2048
T = 4096
CAP = int(TOP_K * T / E * 1.25)

NF = 4
BF = D_FF // NF

BT = 256
NB = T // BT

NC, NS = 2, 16
NW = NC * NS
TPW = T // NW
BUF_ROWS = E * CAP + 8


def _router_body(x_ref, wr_ref, ints_ref, gates_ref, carry_ref):
    i = pl.program_id(0)
    logits = jnp.dot(x_ref[...], wr_ref[...], preferred_element_type=jnp.float32)
    iota_e = lax.broadcasted_iota(jnp.int32, (BT, E), 1)
    m1 = jnp.max(logits, axis=1, keepdims=True)
    am1 = jnp.min(jnp.where(logits == m1, iota_e, E), axis=1, keepdims=True)
    masked = jnp.where(iota_e == am1, -jnp.inf, logits)
    m2 = jnp.max(masked, axis=1, keepdims=True)
    am2 = jnp.min(jnp.where(masked == m2, iota_e, E), axis=1, keepdims=True)
    g0 = 1.0 / (1.0 + jnp.exp(m2 - m1))
    g1 = 1.0 - g0

    oh0 = (iota_e == am1).astype(jnp.float32)
    oh1 = (iota_e == am2).astype(jnp.float32)
    ohsum = oh0 + oh1
    r_i = lax.broadcasted_iota(jnp.int32, (BT, BT), 0)
    c_i = lax.broadcasted_iota(jnp.int32, (BT, BT), 1)
    tri = jnp.where(c_i < r_i, 1.0, 0.0).astype(jnp.float32)
    excl = jnp.dot(tri, ohsum, preferred_element_type=jnp.float32)

    @pl.when(i == 0)
    def _init():
        carry_ref[...] = jnp.zeros((1, E), jnp.float32)

    carry = carry_ref[...]
    base = carry + excl
    pos0 = jnp.sum(oh0 * base, axis=1, keepdims=True).astype(jnp.int32)
    pos1 = jnp.sum(oh1 * (base + oh0), axis=1, keepdims=True).astype(jnp.int32)
    carry_ref[...] = carry + jnp.sum(ohsum, axis=0, keepdims=True)

    v0 = pos0 < CAP
    v1 = pos1 < CAP
    disp0 = jnp.where(v0, am1 * CAP + pos0, E * CAP)
    disp1 = jnp.where(v1, am2 * CAP + pos1, E * CAP)
    slot0 = am1 * CAP + jnp.minimum(pos0, CAP - 1)
    slot1 = am2 * CAP + jnp.minimum(pos1, CAP - 1)
    w0 = jnp.where(v0, g0, 0.0)
    w1 = jnp.where(v1, g1, 0.0)
    ints_ref[...] = jnp.concatenate([disp0, disp1, slot0, slot1], axis=1)
    gates_ref[...] = jnp.concatenate([w0, w1], axis=1)


def _router(x, Wr):
    return pl.pallas_call(
        _router_body,
        grid=(NB,),
        in_specs=[
            pl.BlockSpec((BT, D_MODEL), lambda i: (i, 0)),
            pl.BlockSpec((D_MODEL, E), lambda i: (0, 0)),
        ],
        out_specs=[
            pl.BlockSpec((BT, 4), lambda i: (i, 0)),
            pl.BlockSpec((BT, 2), lambda i: (i, 0)),
        ],
        out_shape=[
            jax.ShapeDtypeStruct((T, 4), jnp.int32),
            jax.ShapeDtypeStruct((T, 2), jnp.float32),
        ],
        scratch_shapes=[pltpu.VMEM((1, E), jnp.float32)],
        compiler_params=pltpu.CompilerParams(
            dimension_semantics=("arbitrary",),
        ),
    )(x, Wr)


_SC_MESH = plsc.VectorSubcoreMesh(core_axis_name="c", subcore_axis_name="s")
_DCH = 64


@functools.partial(
    pl.kernel,
    out_type=jax.ShapeDtypeStruct((BUF_ROWS, D_MODEL), jnp.float32),
    mesh=_SC_MESH,
    scratch_types=[
        pltpu.VMEM((2, _DCH), jnp.int32),
        pltpu.VMEM((_DCH, D_MODEL), jnp.float32),
        pltpu.SemaphoreType.DMA,
    ],
)
def _dispatch(x_hbm, ints_hbm, buf_hbm, idx_v, rows_v, sem):
    wid = lax.axis_index("s") * NC + lax.axis_index("c")
    for ch in range(TPW // _DCH):
        base = wid * TPW + ch * _DCH
        pltpu.sync_copy(x_hbm.at[pl.ds(base, _DCH)], rows_v)
        pltpu.sync_copy(ints_hbm.at[0, pl.ds(base, _DCH)], idx_v.at[0])
        pltpu.sync_copy(ints_hbm.at[1, pl.ds(base, _DCH)], idx_v.at[1])
        c0 = pltpu.async_copy(rows_v, buf_hbm.at[idx_v.at[0]], sem)
        c1 = pltpu.async_copy(rows_v, buf_hbm.at[idx_v.at[1]], sem)
        c0.wait()
        c1.wait()


def _ffn_body(buf_ref, wa_ref, wb_ref, wd_ref, out_ref):
    f = pl.program_id(1)
    xb = buf_ref[...].astype(jnp.bfloat16)
    wa = wa_ref[0].astype(jnp.bfloat16)
    wb = wb_ref[0].astype(jnp.bfloat16)
    a = jnp.dot(xb, wa, preferred_element_type=jnp.float32)
    b = jnp.dot(xb, wb, preferred_element_type=jnp.float32)
    h = (a * jax.nn.sigmoid(a) * b).astype(jnp.bfloat16)
    wd = wd_ref[0].astype(jnp.bfloat16)
    contrib = jnp.dot(h, wd, preferred_element_type=jnp.float32)

    @pl.when(f == 0)
    def _init():
        out_ref[...] = contrib

    @pl.when(f > 0)
    def _acc():
        out_ref[...] += contrib


def _ffn(buf, w_up, w_down):
    return pl.pallas_call(
        _ffn_body,
        grid=(E, NF),
        in_specs=[
            pl.BlockSpec((CAP, D_MODEL), lambda e, f: (e, 0)),
            pl.BlockSpec((1, D_MODEL, BF), lambda e, f: (e, 0, f)),
            pl.BlockSpec((1, D_MODEL, BF), lambda e, f: (e, 0, NF + f)),
            pl.BlockSpec((1, BF, D_MODEL), lambda e, f: (e, f, 0)),
        ],
        out_specs=pl.BlockSpec((CAP, D_MODEL), lambda e, f: (e, 0)),
        out_shape=jax.ShapeDtypeStruct((E * CAP, D_MODEL), jnp.float32),
        compiler_params=pltpu.CompilerParams(
            dimension_semantics=("arbitrary", "arbitrary"),
        ),
    )(buf, w_up, w_up, w_down)


_CCH = 32


@functools.partial(
    pl.kernel,
    out_type=jax.ShapeDtypeStruct((T, D_MODEL), jnp.float32),
    mesh=_SC_MESH,
    scratch_types=[
        pltpu.VMEM((2, _CCH), jnp.int32),
        pltpu.VMEM((2, _CCH, 16), jnp.float32),
        pltpu.VMEM((_CCH, D_MODEL), jnp.float32),
        pltpu.VMEM((_CCH, D_MODEL), jnp.float32),
        pltpu.SemaphoreType.DMA,
    ],
)
def _combine(out_hbm, ints_hbm, g16_hbm, y_hbm, idx_v, w_v, r0_v, r1_v, sem):
    wid = lax.axis_index("s") * NC + lax.axis_index("c")
    for ch in range(TPW // _CCH):
        base = wid * TPW + ch * _CCH
        pltpu.sync_copy(ints_hbm.at[2, pl.ds(base, _CCH)], idx_v.at[0])
        pltpu.sync_copy(ints_hbm.at[3, pl.ds(base, _CCH)], idx_v.at[1])
        pltpu.sync_copy(g16_hbm.at[0, pl.ds(base, _CCH)], w_v.at[0])
        pltpu.sync_copy(g16_hbm.at[1, pl.ds(base, _CCH)], w_v.at[1])
        c0 = pltpu.async_copy(out_hbm.at[idx_v.at[0]], r0_v, sem)
        c1 = pltpu.async_copy(out_hbm.at[idx_v.at[1]], r1_v, sem)
        c0.wait()
        c1.wait()

        def row_body(r, carry):
            w0 = w_v[0, r]
            w1 = w_v[1, r]

            def vec_body(v, c2):
                sl = pl.ds(v * 16, 16)
                r0_v[r, sl] = w0 * r0_v[r, sl] + w1 * r1_v[r, sl]
                return c2

            lax.fori_loop(0, D_MODEL // 16, vec_body, 0)
            return carry

        lax.fori_loop(0, _CCH, row_body, 0)
        pltpu.sync_copy(r0_v, y_hbm.at[pl.ds(base, _CCH)])


def kernel(x, Wr, w_up, w_down):
    ints, gates = _router(x, Wr)
    ints_t = ints.T
    g16 = jnp.broadcast_to(gates.T[:, :, None], (TOP_K, T, 16))
    buf = _dispatch(x, ints_t)
    out = _ffn(buf, w_up, w_down)
    y = _combine(out, ints_t, g16)
    return y


# speedup vs baseline: 1.7413x; 1.1097x over previous
import functools

import jax
import jax.numpy as jnp
from jax import lax
from jax.experimental import pallas as pl
from jax.experimental.pallas import tpu as pltpu
from jax.experimental.pallas import tpu_sc as plsc

E = 8
TOP_K = 2
D_MODEL = 1024
D_FF = 2048
T = 4096
CAP = int(TOP_K * T / E * 1.25)

NF = 4
BF = D_FF // NF

BT = 256
NB = T // BT

NC, NS = 2, 16
NW = NC * NS
TPW = T // NW
BUF_ROWS = E * CAP + 8


def _router_body(x_ref, wr_ref, ints_ref, gates_ref, carry_ref):
    i = pl.program_id(0)
    logits = jnp.dot(x_ref[...], wr_ref[...], preferred_element_type=jnp.float32)
    iota_e = lax.broadcasted_iota(jnp.int32, (BT, E), 1)
    m1 = jnp.max(logits, axis=1, keepdims=True)
    am1 = jnp.min(jnp.where(logits == m1, iota_e, E), axis=1, keepdims=True)
    masked = jnp.where(iota_e == am1, -jnp.inf, logits)
    m2 = jnp.max(masked, axis=1, keepdims=True)
    am2 = jnp.min(jnp.where(masked == m2, iota_e, E), axis=1, keepdims=True)
    g0 = 1.0 / (1.0 + jnp.exp(m2 - m1))
    g1 = 1.0 - g0

    oh0 = (iota_e == am1).astype(jnp.float32)
    oh1 = (iota_e == am2).astype(jnp.float32)
    ohsum = oh0 + oh1
    r_i = lax.broadcasted_iota(jnp.int32, (BT, BT), 0)
    c_i = lax.broadcasted_iota(jnp.int32, (BT, BT), 1)
    tri = jnp.where(c_i < r_i, 1.0, 0.0).astype(jnp.float32)
    excl = jnp.dot(tri, ohsum, preferred_element_type=jnp.float32)

    @pl.when(i == 0)
    def _init():
        carry_ref[...] = jnp.zeros((1, E), jnp.float32)

    carry = carry_ref[...]
    base = carry + excl
    pos0 = jnp.sum(oh0 * base, axis=1, keepdims=True).astype(jnp.int32)
    pos1 = jnp.sum(oh1 * (base + oh0), axis=1, keepdims=True).astype(jnp.int32)
    carry_ref[...] = carry + jnp.sum(ohsum, axis=0, keepdims=True)

    v0 = pos0 < CAP
    v1 = pos1 < CAP
    disp0 = jnp.where(v0, am1 * CAP + pos0, E * CAP)
    disp1 = jnp.where(v1, am2 * CAP + pos1, E * CAP)
    slot0 = am1 * CAP + jnp.minimum(pos0, CAP - 1)
    slot1 = am2 * CAP + jnp.minimum(pos1, CAP - 1)
    w0 = jnp.where(v0, g0, 0.0)
    w1 = jnp.where(v1, g1, 0.0)
    ints_ref[...] = jnp.concatenate([disp0, disp1, slot0, slot1], axis=1)
    gates_ref[...] = jnp.concatenate([w0, w1], axis=1)


def _router(x, Wr):
    return pl.pallas_call(
        _router_body,
        grid=(NB,),
        in_specs=[
            pl.BlockSpec((BT, D_MODEL), lambda i: (i, 0)),
            pl.BlockSpec((D_MODEL, E), lambda i: (0, 0)),
        ],
        out_specs=[
            pl.BlockSpec((BT, 4), lambda i: (i, 0)),
            pl.BlockSpec((BT, 2), lambda i: (i, 0)),
        ],
        out_shape=[
            jax.ShapeDtypeStruct((T, 4), jnp.int32),
            jax.ShapeDtypeStruct((T, 2), jnp.float32),
        ],
        scratch_shapes=[pltpu.VMEM((1, E), jnp.float32)],
        compiler_params=pltpu.CompilerParams(
            dimension_semantics=("arbitrary",),
        ),
    )(x, Wr)


_SC_MESH = plsc.VectorSubcoreMesh(core_axis_name="c", subcore_axis_name="s")
_DCH = 64


@functools.partial(
    pl.kernel,
    out_type=jax.ShapeDtypeStruct((BUF_ROWS, D_MODEL), jnp.float32),
    mesh=_SC_MESH,
    scratch_types=[
        pltpu.VMEM((2, _DCH), jnp.int32),
        pltpu.VMEM((_DCH, D_MODEL), jnp.float32),
        pltpu.SemaphoreType.DMA,
    ],
)
def _dispatch(x_hbm, ints_hbm, buf_hbm, idx_v, rows_v, sem):
    wid = lax.axis_index("s") * NC + lax.axis_index("c")
    for ch in range(TPW // _DCH):
        base = wid * TPW + ch * _DCH
        pltpu.sync_copy(x_hbm.at[pl.ds(base, _DCH)], rows_v)
        pltpu.sync_copy(ints_hbm.at[0, pl.ds(base, _DCH)], idx_v.at[0])
        pltpu.sync_copy(ints_hbm.at[1, pl.ds(base, _DCH)], idx_v.at[1])
        c0 = pltpu.async_copy(rows_v, buf_hbm.at[idx_v.at[0]], sem)
        c1 = pltpu.async_copy(rows_v, buf_hbm.at[idx_v.at[1]], sem)
        c0.wait()
        c1.wait()


def _ffn_body(buf_ref, wa_ref, wb_ref, wd_ref, out_ref):
    f = pl.program_id(1)
    xb = buf_ref[...].astype(jnp.bfloat16)
    wab = jnp.concatenate(
        [wa_ref[0].astype(jnp.bfloat16), wb_ref[0].astype(jnp.bfloat16)], axis=1
    )
    ab = jnp.dot(xb, wab, preferred_element_type=jnp.float32)
    a = ab[:, :BF]
    b = ab[:, BF:]
    h = (a * jax.nn.sigmoid(a) * b).astype(jnp.bfloat16)
    wd = wd_ref[0].astype(jnp.bfloat16)
    contrib = jnp.dot(h, wd, preferred_element_type=jnp.float32)

    @pl.when(f == 0)
    def _init():
        out_ref[...] = contrib

    @pl.when(f > 0)
    def _acc():
        out_ref[...] += contrib


def _ffn(buf, w_up, w_down):
    return pl.pallas_call(
        _ffn_body,
        grid=(E, NF),
        in_specs=[
            pl.BlockSpec((CAP, D_MODEL), lambda e, f: (e, 0)),
            pl.BlockSpec((1, D_MODEL, BF), lambda e, f: (e, 0, f)),
            pl.BlockSpec((1, D_MODEL, BF), lambda e, f: (e, 0, NF + f)),
            pl.BlockSpec((1, BF, D_MODEL), lambda e, f: (e, f, 0)),
        ],
        out_specs=pl.BlockSpec((CAP, D_MODEL), lambda e, f: (e, 0)),
        out_shape=jax.ShapeDtypeStruct((E * CAP, D_MODEL), jnp.float32),
        compiler_params=pltpu.CompilerParams(
            dimension_semantics=("arbitrary", "arbitrary"),
        ),
    )(buf, w_up, w_up, w_down)


_CCH = 32


@functools.partial(
    pl.kernel,
    out_type=jax.ShapeDtypeStruct((T, D_MODEL), jnp.float32),
    mesh=_SC_MESH,
    scratch_types=[
        pltpu.VMEM((2, _CCH), jnp.int32),
        pltpu.VMEM((2, _CCH, 16), jnp.float32),
        pltpu.VMEM((_CCH, D_MODEL), jnp.float32),
        pltpu.VMEM((_CCH, D_MODEL), jnp.float32),
        pltpu.SemaphoreType.DMA,
    ],
)
def _combine(out_hbm, ints_hbm, g16_hbm, y_hbm, idx_v, w_v, r0_v, r1_v, sem):
    wid = lax.axis_index("s") * NC + lax.axis_index("c")
    for ch in range(TPW // _CCH):
        base = wid * TPW + ch * _CCH
        pltpu.sync_copy(ints_hbm.at[2, pl.ds(base, _CCH)], idx_v.at[0])
        pltpu.sync_copy(ints_hbm.at[3, pl.ds(base, _CCH)], idx_v.at[1])
        pltpu.sync_copy(g16_hbm.at[0, pl.ds(base, _CCH)], w_v.at[0])
        pltpu.sync_copy(g16_hbm.at[1, pl.ds(base, _CCH)], w_v.at[1])
        c0 = pltpu.async_copy(out_hbm.at[idx_v.at[0]], r0_v, sem)
        c1 = pltpu.async_copy(out_hbm.at[idx_v.at[1]], r1_v, sem)
        c0.wait()
        c1.wait()

        def row_body(r, carry):
            w0 = w_v[0, r]
            w1 = w_v[1, r]
            for v in range(D_MODEL // 16):
                sl = pl.ds(v * 16, 16)
                r0_v[r, sl] = w0 * r0_v[r, sl] + w1 * r1_v[r, sl]
            return carry

        lax.fori_loop(0, _CCH, row_body, 0)
        pltpu.sync_copy(r0_v, y_hbm.at[pl.ds(base, _CCH)])


def kernel(x, Wr, w_up, w_down):
    ints, gates = _router(x, Wr)
    ints_t = ints.T
    g16 = jnp.broadcast_to(gates.T[:, :, None], (TOP_K, T, 16))
    buf = _dispatch(x, ints_t)
    out = _ffn(buf, w_up, w_down)
    y = _combine(out, ints_t, g16)
    return y


# speedup vs baseline: 1.7772x; 1.0206x over previous
import functools

import jax
import jax.numpy as jnp
from jax import lax
from jax.experimental import pallas as pl
from jax.experimental.pallas import tpu as pltpu
from jax.experimental.pallas import tpu_sc as plsc

E = 8
TOP_K = 2
D_MODEL = 1024
D_FF = 2048
T = 4096
CAP = int(TOP_K * T / E * 1.25)

NF = 4
BF = D_FF // NF

BT = 256
NB = T // BT

NC, NS = 2, 16
NW = NC * NS
TPW = T // NW
BUF_ROWS = E * CAP + 8


def _router_body(x_ref, wr_ref, tri_ref, meta_ref, gates_ref, carry_ref):
    i = pl.program_id(0)
    logits = jnp.dot(x_ref[...], wr_ref[...], preferred_element_type=jnp.float32)
    iota_e = lax.broadcasted_iota(jnp.int32, (BT, E), 1)
    m1 = jnp.max(logits, axis=1, keepdims=True)
    am1 = jnp.min(jnp.where(logits == m1, iota_e, E), axis=1, keepdims=True)
    masked = jnp.where(iota_e == am1, -jnp.inf, logits)
    m2 = jnp.max(masked, axis=1, keepdims=True)
    am2 = jnp.min(jnp.where(masked == m2, iota_e, E), axis=1, keepdims=True)
    g0 = 1.0 / (1.0 + jnp.exp(m2 - m1))
    g1 = 1.0 - g0

    oh0 = (iota_e == am1).astype(jnp.float32)
    oh1 = (iota_e == am2).astype(jnp.float32)
    ohsum = oh0 + oh1
    excl = jnp.dot(tri_ref[...], ohsum, preferred_element_type=jnp.float32)

    @pl.when(i == 0)
    def _init():
        carry_ref[...] = jnp.zeros((1, E), jnp.float32)

    carry = carry_ref[...]
    base = carry + excl
    pos0 = jnp.sum(oh0 * base, axis=1, keepdims=True).astype(jnp.int32)
    pos1 = jnp.sum(oh1 * (base + oh0), axis=1, keepdims=True).astype(jnp.int32)
    carry_ref[...] = carry + jnp.sum(ohsum, axis=0, keepdims=True)

    v0 = pos0 < CAP
    v1 = pos1 < CAP
    disp0 = jnp.where(v0, am1 * CAP + pos0, E * CAP)
    disp1 = jnp.where(v1, am2 * CAP + pos1, E * CAP)
    slot0 = am1 * CAP + jnp.minimum(pos0, CAP - 1)
    slot1 = am2 * CAP + jnp.minimum(pos1, CAP - 1)
    w0 = jnp.where(v0, g0, 0.0)
    w1 = jnp.where(v1, g1, 0.0)
    block = jnp.concatenate(
        [disp0, disp1, slot0, slot1, jnp.zeros((BT, 4), jnp.int32)], axis=1
    )
    meta_ref[...] = block.T
    gblock = jnp.concatenate([w0, w1, jnp.zeros((BT, 6), jnp.float32)], axis=1)
    gates_ref[...] = gblock.T


def _router(x, Wr, tri):
    return pl.pallas_call(
        _router_body,
        grid=(NB,),
        in_specs=[
            pl.BlockSpec((BT, D_MODEL), lambda i: (i, 0)),
            pl.BlockSpec((D_MODEL, E), lambda i: (0, 0)),
            pl.BlockSpec((BT, BT), lambda i: (0, 0)),
        ],
        out_specs=[
            pl.BlockSpec((8, BT), lambda i: (0, i)),
            pl.BlockSpec((8, BT), lambda i: (0, i)),
        ],
        out_shape=[
            jax.ShapeDtypeStruct((8, T), jnp.int32),
            jax.ShapeDtypeStruct((8, T), jnp.float32),
        ],
        scratch_shapes=[pltpu.VMEM((1, E), jnp.float32)],
        compiler_params=pltpu.CompilerParams(
            dimension_semantics=("arbitrary",),
        ),
    )(x, Wr, tri)


_SC_MESH = plsc.VectorSubcoreMesh(core_axis_name="c", subcore_axis_name="s")
_DCH = 64


@functools.partial(
    pl.kernel,
    out_type=jax.ShapeDtypeStruct((BUF_ROWS, D_MODEL), jnp.float32),
    mesh=_SC_MESH,
    scratch_types=[
        pltpu.VMEM((2, _DCH), jnp.int32),
        pltpu.VMEM((_DCH, D_MODEL), jnp.float32),
        pltpu.SemaphoreType.DMA,
    ],
)
def _dispatch(x_hbm, meta_hbm, buf_hbm, idx_v, rows_v, sem):
    wid = lax.axis_index("s") * NC + lax.axis_index("c")
    for ch in range(TPW // _DCH):
        base = wid * TPW + ch * _DCH
        pltpu.sync_copy(x_hbm.at[pl.ds(base, _DCH)], rows_v)
        pltpu.sync_copy(meta_hbm.at[0, pl.ds(base, _DCH)], idx_v.at[0])
        pltpu.sync_copy(meta_hbm.at[1, pl.ds(base, _DCH)], idx_v.at[1])
        c0 = pltpu.async_copy(rows_v, buf_hbm.at[idx_v.at[0]], sem)
        c1 = pltpu.async_copy(rows_v, buf_hbm.at[idx_v.at[1]], sem)
        c0.wait()
        c1.wait()


def _ffn_body(buf_ref, wa_ref, wb_ref, wd_ref, out_ref):
    f = pl.program_id(1)
    xb = buf_ref[...].astype(jnp.bfloat16)
    wab = jnp.concatenate(
        [wa_ref[0].astype(jnp.bfloat16), wb_ref[0].astype(jnp.bfloat16)], axis=1
    )
    ab = jnp.dot(xb, wab, preferred_element_type=jnp.float32)
    a = ab[:, :BF]
    b = ab[:, BF:]
    h = (a * jax.nn.sigmoid(a) * b).astype(jnp.bfloat16)
    wd = wd_ref[0].astype(jnp.bfloat16)
    contrib = jnp.dot(h, wd, preferred_element_type=jnp.float32)

    @pl.when(f == 0)
    def _init():
        out_ref[...] = contrib

    @pl.when(f > 0)
    def _acc():
        out_ref[...] += contrib


def _ffn(buf, w_up, w_down):
    return pl.pallas_call(
        _ffn_body,
        grid=(E, NF),
        in_specs=[
            pl.BlockSpec((CAP, D_MODEL), lambda e, f: (e, 0)),
            pl.BlockSpec((1, D_MODEL, BF), lambda e, f: (e, 0, f)),
            pl.BlockSpec((1, D_MODEL, BF), lambda e, f: (e, 0, NF + f)),
            pl.BlockSpec((1, BF, D_MODEL), lambda e, f: (e, f, 0)),
        ],
        out_specs=pl.BlockSpec((CAP, D_MODEL), lambda e, f: (e, 0)),
        out_shape=jax.ShapeDtypeStruct((E * CAP, D_MODEL), jnp.float32),
        compiler_params=pltpu.CompilerParams(
            dimension_semantics=("arbitrary", "arbitrary"),
        ),
    )(buf, w_up, w_up, w_down)


_CCH = 16
_NCH = TPW // _CCH


@functools.partial(
    pl.kernel,
    out_type=jax.ShapeDtypeStruct((T, D_MODEL), jnp.float32),
    mesh=_SC_MESH,
    scratch_types=[
        pltpu.VMEM((2, TPW), jnp.int32),
        pltpu.VMEM((2, TPW, 16), jnp.float32),
        pltpu.VMEM((2, _CCH, D_MODEL), jnp.float32),
        pltpu.VMEM((2, _CCH, D_MODEL), jnp.float32),
        pltpu.SemaphoreType.DMA,
        pltpu.SemaphoreType.DMA,
        pltpu.SemaphoreType.DMA,
    ],
)
def _combine(out_hbm, meta_hbm, g16_hbm, y_hbm, meta_v, gates_v, bufa, bufb,
             gsema, gsemb, ysem):
    wid = lax.axis_index("s") * NC + lax.axis_index("c")
    w0 = wid * TPW
    pltpu.sync_copy(meta_hbm.at[pl.ds(2, 2), pl.ds(w0, TPW)], meta_v)
    pltpu.sync_copy(g16_hbm.at[0, pl.ds(w0, TPW)], gates_v.at[0])
    pltpu.sync_copy(g16_hbm.at[1, pl.ds(w0, TPW)], gates_v.at[1])

    bufs = (bufa, bufb)
    sems = (gsema, gsemb)

    def fire(ch):
        b = bufs[ch % 2]
        s = sems[ch % 2]
        c0 = pltpu.async_copy(
            out_hbm.at[meta_v.at[0, pl.ds(ch * _CCH, _CCH)]], b.at[0], s)
        c1 = pltpu.async_copy(
            out_hbm.at[meta_v.at[1, pl.ds(ch * _CCH, _CCH)]], b.at[1], s)
        return (c0, c1)

    def compute(ch):
        b = bufs[ch % 2]

        def row_body(r, carry):
            t = ch * _CCH + r
            wv0 = gates_v[0, t]
            wv1 = gates_v[1, t]
            for v in range(D_MODEL // 16):
                sl = pl.ds(v * 16, 16)
                b[0, r, sl] = wv0 * b[0, r, sl] + wv1 * b[1, r, sl]
            return carry

        lax.fori_loop(0, _CCH, row_body, 0)
        return pltpu.async_copy(b.at[0], y_hbm.at[pl.ds(w0 + ch * _CCH, _CCH)], ysem)

    pending_g = {0: fire(0)}
    pending_y = {}
    for ch in range(_NCH):
        if ch + 1 < _NCH:
            if ch - 1 in pending_y:
                pending_y.pop(ch - 1).wait()
            pending_g[ch + 1] = fire(ch + 1)
        g0, g1 = pending_g.pop(ch)
        g0.wait()
        g1.wait()
        pending_y[ch] = compute(ch)
    for c in pending_y.values():
        c.wait()


def kernel(x, Wr, w_up, w_down):
    r_i = jnp.arange(BT, dtype=jnp.int32)[:, None]
    c_i = jnp.arange(BT, dtype=jnp.int32)[None, :]
    tri = (c_i < r_i).astype(jnp.float32)
    meta, gates = _router(x, Wr, tri)
    g16 = jnp.broadcast_to(gates[:2, :, None], (TOP_K, T, 16))
    buf = _dispatch(x, meta)
    out = _ffn(buf, w_up, w_down)
    y = _combine(out, meta, g16)
    return y


# speedup vs baseline: 1.8216x; 1.0250x over previous
import functools

import jax
import jax.numpy as jnp
from jax import lax
from jax.experimental import pallas as pl
from jax.experimental.pallas import tpu as pltpu
from jax.experimental.pallas import tpu_sc as plsc

E = 8
TOP_K = 2
D_MODEL = 1024
D_FF = 2048
T = 4096
CAP = int(TOP_K * T / E * 1.25)

NF = 2
BF = D_FF // NF

BT = 256
NB = T // BT

NC, NS = 2, 16
NW = NC * NS
TPW = T // NW
BUF_ROWS = E * CAP + 8


def _router_body(x_ref, wr_ref, tri_ref, meta_ref, gates_ref, carry_ref):
    i = pl.program_id(0)
    logits = jnp.dot(x_ref[...], wr_ref[...], preferred_element_type=jnp.float32)
    iota_e = lax.broadcasted_iota(jnp.int32, (BT, E), 1)
    m1 = jnp.max(logits, axis=1, keepdims=True)
    am1 = jnp.min(jnp.where(logits == m1, iota_e, E), axis=1, keepdims=True)
    masked = jnp.where(iota_e == am1, -jnp.inf, logits)
    m2 = jnp.max(masked, axis=1, keepdims=True)
    am2 = jnp.min(jnp.where(masked == m2, iota_e, E), axis=1, keepdims=True)
    g0 = 1.0 / (1.0 + jnp.exp(m2 - m1))
    g1 = 1.0 - g0

    oh0 = (iota_e == am1).astype(jnp.float32)
    oh1 = (iota_e == am2).astype(jnp.float32)
    ohsum = oh0 + oh1
    excl = jnp.dot(tri_ref[...], ohsum, preferred_element_type=jnp.float32)

    @pl.when(i == 0)
    def _init():
        carry_ref[...] = jnp.zeros((1, E), jnp.float32)

    carry = carry_ref[...]
    base = carry + excl
    pos0 = jnp.sum(oh0 * base, axis=1, keepdims=True).astype(jnp.int32)
    pos1 = jnp.sum(oh1 * (base + oh0), axis=1, keepdims=True).astype(jnp.int32)
    carry_ref[...] = carry + jnp.sum(ohsum, axis=0, keepdims=True)

    v0 = pos0 < CAP
    v1 = pos1 < CAP
    disp0 = jnp.where(v0, am1 * CAP + pos0, E * CAP)
    disp1 = jnp.where(v1, am2 * CAP + pos1, E * CAP)
    slot0 = am1 * CAP + jnp.minimum(pos0, CAP - 1)
    slot1 = am2 * CAP + jnp.minimum(pos1, CAP - 1)
    w0 = jnp.where(v0, g0, 0.0)
    w1 = jnp.where(v1, g1, 0.0)
    block = jnp.concatenate(
        [disp0, disp1, slot0, slot1, jnp.zeros((BT, 4), jnp.int32)], axis=1
    )
    meta_ref[...] = block.T
    gt = jnp.concatenate([w0, w1], axis=1).T
    gates_ref[...] = jnp.broadcast_to(gt[:, :, None], (TOP_K, BT, 16))


def _router(x, Wr, tri):
    return pl.pallas_call(
        _router_body,
        grid=(NB,),
        in_specs=[
            pl.BlockSpec((BT, D_MODEL), lambda i: (i, 0)),
            pl.BlockSpec((D_MODEL, E), lambda i: (0, 0)),
            pl.BlockSpec((BT, BT), lambda i: (0, 0)),
        ],
        out_specs=[
            pl.BlockSpec((8, BT), lambda i: (0, i)),
            pl.BlockSpec((TOP_K, BT, 16), lambda i: (0, i, 0)),
        ],
        out_shape=[
            jax.ShapeDtypeStruct((8, T), jnp.int32),
            jax.ShapeDtypeStruct((TOP_K, T, 16), jnp.float32),
        ],
        scratch_shapes=[pltpu.VMEM((1, E), jnp.float32)],
        compiler_params=pltpu.CompilerParams(
            dimension_semantics=("arbitrary",),
        ),
    )(x, Wr, tri)


_SC_MESH = plsc.VectorSubcoreMesh(core_axis_name="c", subcore_axis_name="s")
_DCH = 64


@functools.partial(
    pl.kernel,
    out_type=jax.ShapeDtypeStruct((BUF_ROWS, D_MODEL), jnp.float32),
    mesh=_SC_MESH,
    scratch_types=[
        pltpu.VMEM((2, _DCH), jnp.int32),
        pltpu.VMEM((_DCH, D_MODEL), jnp.float32),
        pltpu.SemaphoreType.DMA,
    ],
)
def _dispatch(x_hbm, meta_hbm, buf_hbm, idx_v, rows_v, sem):
    wid = lax.axis_index("s") * NC + lax.axis_index("c")
    for ch in range(TPW // _DCH):
        base = wid * TPW + ch * _DCH
        pltpu.sync_copy(x_hbm.at[pl.ds(base, _DCH)], rows_v)
        pltpu.sync_copy(meta_hbm.at[0, pl.ds(base, _DCH)], idx_v.at[0])
        pltpu.sync_copy(meta_hbm.at[1, pl.ds(base, _DCH)], idx_v.at[1])
        c0 = pltpu.async_copy(rows_v, buf_hbm.at[idx_v.at[0]], sem)
        c1 = pltpu.async_copy(rows_v, buf_hbm.at[idx_v.at[1]], sem)
        c0.wait()
        c1.wait()


def _ffn_body(buf_ref, wa_ref, wb_ref, wd_ref, out_ref):
    f = pl.program_id(1)
    xb = buf_ref[...].astype(jnp.bfloat16)
    wab = jnp.concatenate(
        [wa_ref[0].astype(jnp.bfloat16), wb_ref[0].astype(jnp.bfloat16)], axis=1
    )
    ab = jnp.dot(xb, wab, preferred_element_type=jnp.float32)
    a = ab[:, :BF]
    b = ab[:, BF:]
    h = (a * jax.nn.sigmoid(a) * b).astype(jnp.bfloat16)
    wd = wd_ref[0].astype(jnp.bfloat16)
    contrib = jnp.dot(h, wd, preferred_element_type=jnp.float32)

    @pl.when(f == 0)
    def _init():
        out_ref[...] = contrib

    @pl.when(f > 0)
    def _acc():
        out_ref[...] += contrib


def _ffn(buf, w_up, w_down):
    return pl.pallas_call(
        _ffn_body,
        grid=(E, NF),
        in_specs=[
            pl.BlockSpec((CAP, D_MODEL), lambda e, f: (e, 0)),
            pl.BlockSpec((1, D_MODEL, BF), lambda e, f: (e, 0, f)),
            pl.BlockSpec((1, D_MODEL, BF), lambda e, f: (e, 0, NF + f)),
            pl.BlockSpec((1, BF, D_MODEL), lambda e, f: (e, f, 0)),
        ],
        out_specs=pl.BlockSpec((CAP, D_MODEL), lambda e, f: (e, 0)),
        out_shape=jax.ShapeDtypeStruct((E * CAP, D_MODEL), jnp.float32),
        compiler_params=pltpu.CompilerParams(
            dimension_semantics=("arbitrary", "arbitrary"),
        ),
    )(buf, w_up, w_up, w_down)


_CCH = 16
_NCH = TPW // _CCH


@functools.partial(
    pl.kernel,
    out_type=jax.ShapeDtypeStruct((T, D_MODEL), jnp.float32),
    mesh=_SC_MESH,
    scratch_types=[
        pltpu.VMEM((2, TPW), jnp.int32),
        pltpu.VMEM((2, TPW, 16), jnp.float32),
        pltpu.VMEM((2, _CCH, D_MODEL), jnp.float32),
        pltpu.VMEM((2, _CCH, D_MODEL), jnp.float32),
        pltpu.SemaphoreType.DMA,
        pltpu.SemaphoreType.DMA,
        pltpu.SemaphoreType.DMA,
    ],
)
def _combine(out_hbm, meta_hbm, g16_hbm, y_hbm, meta_v, gates_v, bufa, bufb,
             gsema, gsemb, ysem):
    wid = lax.axis_index("s") * NC + lax.axis_index("c")
    w0 = wid * TPW
    pltpu.sync_copy(meta_hbm.at[pl.ds(2, 2), pl.ds(w0, TPW)], meta_v)
    pltpu.sync_copy(g16_hbm.at[0, pl.ds(w0, TPW)], gates_v.at[0])
    pltpu.sync_copy(g16_hbm.at[1, pl.ds(w0, TPW)], gates_v.at[1])

    bufs = (bufa, bufb)
    sems = (gsema, gsemb)

    def fire(ch):
        b = bufs[ch % 2]
        s = sems[ch % 2]
        c0 = pltpu.async_copy(
            out_hbm.at[meta_v.at[0, pl.ds(ch * _CCH, _CCH)]], b.at[0], s)
        c1 = pltpu.async_copy(
            out_hbm.at[meta_v.at[1, pl.ds(ch * _CCH, _CCH)]], b.at[1], s)
        return (c0, c1)

    def compute(ch):
        b = bufs[ch % 2]

        def row_body(r, carry):
            t = ch * _CCH + r
            wv0 = gates_v[0, t]
            wv1 = gates_v[1, t]
            for v in range(D_MODEL // 16):
                sl = pl.ds(v * 16, 16)
                b[0, r, sl] = wv0 * b[0, r, sl] + wv1 * b[1, r, sl]
            return carry

        lax.fori_loop(0, _CCH, row_body, 0)
        return pltpu.async_copy(b.at[0], y_hbm.at[pl.ds(w0 + ch * _CCH, _CCH)], ysem)

    pending_g = {0: fire(0)}
    pending_y = {}
    for ch in range(_NCH):
        if ch + 1 < _NCH:
            if ch - 1 in pending_y:
                pending_y.pop(ch - 1).wait()
            pending_g[ch + 1] = fire(ch + 1)
        g0, g1 = pending_g.pop(ch)
        g0.wait()
        g1.wait()
        pending_y[ch] = compute(ch)
    for c in pending_y.values():
        c.wait()


def kernel(x, Wr, w_up, w_down):
    r_i = jnp.arange(BT, dtype=jnp.int32)[:, None]
    c_i = jnp.arange(BT, dtype=jnp.int32)[None, :]
    tri = (c_i < r_i).astype(jnp.float32)
    meta, g16 = _router(x, Wr, tri)
    buf = _dispatch(x, meta)
    out = _ffn(buf, w_up, w_down)
    y = _combine(out, meta, g16)
    return y
